# Initial kernel scaffold; baseline (speedup 1.0000x reference)
import jax
import jax.numpy as jnp
from jax.experimental import pallas as pl


def kernel(x, adj_lists, Wmsg, bmsg, Wq, Wk):
    raise NotImplementedError("write your pallas kernel here")


# speedup vs baseline: 7.6096x; 7.6096x over previous
import functools

import jax
import jax.numpy as jnp
from jax.experimental import pallas as pl


def _proj_kernel(x_ref, w_ref, b_ref, o_ref):
    o_ref[0] = (
        jnp.dot(x_ref[...], w_ref[0], preferred_element_type=jnp.float32)
        + b_ref[0, 0]
    )


def _edge_kernel(gs_ref, gt_ref, ones_ref, m_ref, ex_ref):
    hid = gs_ref.shape[-1] // 4
    a = gs_ref[:, :hid]
    k = gs_ref[:, 3 * hid:]
    b = gt_ref[:, hid:2 * hid]
    q = gt_ref[:, 2 * hid:3 * hid]
    m_ref[...] = jnp.maximum(a + b, 0.0)
    qk = q * k
    sc = jnp.dot(qk, ones_ref[...], preferred_element_type=jnp.float32)
    ex_ref[...] = jnp.exp(sc)


def _weight_kernel(m_ref, ex_ref, dg_ref, exp_ref, o_ref):
    w = ex_ref[...] / dg_ref[...]
    wb = jnp.dot(w, exp_ref[...], preferred_element_type=jnp.float32)
    o_ref[...] = wb * m_ref[...]


def _pick_block(n, candidates):
    for c in candidates:
        if n % c == 0:
            return c
    return n


@jax.jit
def kernel(x, adj_lists, Wmsg, bmsg, Wq, Wk):
    N, HID = x.shape
    ET, E, _ = adj_lists.shape
    H = 4
    D = HID // H
    SCALE = float(D) ** (-0.5)

    W_all = jnp.concatenate(
        [Wmsg[:, :HID, :], Wmsg[:, HID:, :], Wq * SCALE, Wk], axis=2
    )
    bias = jnp.concatenate(
        [jnp.zeros_like(bmsg), bmsg, jnp.zeros_like(bmsg), jnp.zeros_like(bmsg)],
        axis=1,
    ).reshape(ET, 1, 4 * HID)

    BN = _pick_block(N, (1000, 500, 200, 100, 8))
    P = pl.pallas_call(
        _proj_kernel,
        grid=(ET, N // BN),
        in_specs=[
            pl.BlockSpec((BN, HID), lambda e, n: (n, 0)),
            pl.BlockSpec((1, HID, 4 * HID), lambda e, n: (e, 0, 0)),
            pl.BlockSpec((1, 1, 4 * HID), lambda e, n: (e, 0, 0)),
        ],
        out_specs=pl.BlockSpec((1, BN, 4 * HID), lambda e, n: (e, n, 0)),
        out_shape=jax.ShapeDtypeStruct((ET, N, 4 * HID), jnp.float32),
    )(x, W_all, bias)
    P = P.reshape(ET * N, 4 * HID)

    off = (jnp.arange(ET, dtype=jnp.int32) * N)[:, None]
    src_g = (adj_lists[:, :, 0] + off).reshape(-1)
    tgt = adj_lists[:, :, 1].reshape(-1)
    tgt_g = (adj_lists[:, :, 1] + off).reshape(-1)
    gs = jnp.take(P, src_g, axis=0)
    gt = jnp.take(P, tgt_g, axis=0)

    heads = jnp.arange(HID, dtype=jnp.int32) // D
    ones_hd = (heads[:, None] == jnp.arange(H)[None, :]).astype(jnp.float32)

    ET_E = ET * E
    BE = _pick_block(ET_E, (2000, 1000, 500, 200, 8))
    m_all, ex_all = pl.pallas_call(
        _edge_kernel,
        grid=(ET_E // BE,),
        in_specs=[
            pl.BlockSpec((BE, 4 * HID), lambda i: (i, 0)),
            pl.BlockSpec((BE, 4 * HID), lambda i: (i, 0)),
            pl.BlockSpec((HID, H), lambda i: (0, 0)),
        ],
        out_specs=[
            pl.BlockSpec((BE, HID), lambda i: (i, 0)),
            pl.BlockSpec((BE, H), lambda i: (i, 0)),
        ],
        out_shape=[
            jax.ShapeDtypeStruct((ET_E, HID), jnp.float32),
            jax.ShapeDtypeStruct((ET_E, H), jnp.float32),
        ],
    )(gs, gt, ones_hd)

    denom = jax.ops.segment_sum(ex_all, tgt, num_segments=N)
    dg = jnp.take(denom, tgt, axis=0)

    weighted = pl.pallas_call(
        _weight_kernel,
        grid=(ET_E // BE,),
        in_specs=[
            pl.BlockSpec((BE, HID), lambda i: (i, 0)),
            pl.BlockSpec((BE, H), lambda i: (i, 0)),
            pl.BlockSpec((BE, H), lambda i: (i, 0)),
            pl.BlockSpec((H, HID), lambda i: (0, 0)),
        ],
        out_specs=pl.BlockSpec((BE, HID), lambda i: (i, 0)),
        out_shape=jax.ShapeDtypeStruct((ET_E, HID), jnp.float32),
    )(m_all, ex_all, dg, ones_hd.T)

    return jax.ops.segment_sum(weighted, tgt, num_segments=N)
